# baseline (speedup 1.0000x reference)
import functools

import jax
import jax.numpy as jnp
from jax import lax
from jax.experimental import pallas as pl
from jax.experimental.pallas import tpu as pltpu
from jax.experimental.pallas import tpu_sc as plsc

N = 10000
E = 320000
D = 128
H = 128
OUT = 64

NS = 16
CH = 128
K = 160
KI = 16
E_PAD = NS * K * CH
N_PAD = 10240
RPT = N_PAD // NS
PIECES = RPT // CH
BN = 1000
G_BLK = N // BN


def _sc_mesh():
    return plsc.VectorSubcoreMesh(core_axis_name="c", subcore_axis_name="s")


def _sc_dual_segsum(table, src_i, dst_i, zb):
    @functools.partial(
        pl.kernel,
        out_type=jax.ShapeDtypeStruct((2, N_PAD, D), jnp.float32),
        mesh=_sc_mesh(),
        scratch_types=[
            pltpu.VMEM((KI, CH), jnp.int32),
            pltpu.VMEM((KI, CH), jnp.int32),
            pltpu.VMEM((CH, D), jnp.float32),
            pltpu.VMEM((CH, D), jnp.float32),
            pltpu.SemaphoreType.DMA,
            pltpu.SemaphoreType.DMA,
            pltpu.VMEM_SHARED((N_PAD, D), jnp.float32),
        ],
    )
    def k(table_h, src_h, dst_h, zb_h, out_h,
          src_v, dst_v, rows_a, rows_b, sem_a, sem_b, acc_sh):
        c = lax.axis_index("c")
        s = lax.axis_index("s")
        rs = s * RPT

        @pl.loop(0, PIECES)
        def _(p):
            off = rs + p * CH
            pltpu.sync_copy(zb_h.at[pl.ds(off, CH)], acc_sh.at[pl.ds(off, CH)])

        plsc.subcore_barrier()

        @pl.loop(0, K // KI)
        def _(o):
            pltpu.sync_copy(src_h.at[c, pl.ds(s * K + o * KI, KI)], src_v)
            pltpu.sync_copy(dst_h.at[c, pl.ds(s * K + o * KI, KI)], dst_v)
            pltpu.async_copy(table_h.at[src_v.at[0]], rows_a, sem_a)

            @pl.loop(0, KI // 2)
            def _(jj):
                j0 = 2 * jj
                pltpu.make_async_copy(table_h.at[src_v.at[j0]], rows_a, sem_a).wait()
                pltpu.async_copy(table_h.at[src_v.at[j0 + 1]], rows_b, sem_b)
                pltpu.sync_copy(rows_a, acc_sh.at[dst_v.at[j0]], add=True)
                pltpu.make_async_copy(table_h.at[src_v.at[j0 + 1]], rows_b, sem_b).wait()

                @pl.when(jj < KI // 2 - 1)
                def _():
                    pltpu.async_copy(table_h.at[src_v.at[j0 + 2]], rows_a, sem_a)

                pltpu.sync_copy(rows_b, acc_sh.at[dst_v.at[j0 + 1]], add=True)

        plsc.subcore_barrier()

        @pl.loop(0, PIECES)
        def _(p):
            off = rs + p * CH
            pltpu.sync_copy(acc_sh.at[pl.ds(off, CH)], out_h.at[c, pl.ds(off, CH)])

    return k(table, src_i, dst_i, zb)


def _sc_dual_deg(dst_i, zb, ones_h):
    @functools.partial(
        pl.kernel,
        out_type=jax.ShapeDtypeStruct((2, N_PAD, D), jnp.float32),
        mesh=_sc_mesh(),
        scratch_types=[
            pltpu.VMEM((KI, CH), jnp.int32),
            pltpu.VMEM((CH, D), jnp.float32),
            pltpu.VMEM_SHARED((N_PAD, D), jnp.float32),
        ],
    )
    def k(dst_h, zb_h, ones_hh, out_h, dst_v, ones_v, deg_sh):
        c = lax.axis_index("c")
        s = lax.axis_index("s")
        rs = s * RPT

        @pl.loop(0, PIECES)
        def _(p):
            off = rs + p * CH
            pltpu.sync_copy(zb_h.at[pl.ds(off, CH)], deg_sh.at[pl.ds(off, CH)])

        pltpu.sync_copy(ones_hh, ones_v)
        plsc.subcore_barrier()

        @pl.loop(0, K // KI)
        def _(o):
            pltpu.sync_copy(dst_h.at[c, pl.ds(s * K + o * KI, KI)], dst_v)

            @pl.loop(0, KI)
            def _(j):
                pltpu.sync_copy(ones_v, deg_sh.at[dst_v.at[j]], add=True)

        plsc.subcore_barrier()

        @pl.loop(0, PIECES)
        def _(p):
            off = rs + p * CH
            pltpu.sync_copy(deg_sh.at[pl.ds(off, CH)], out_h.at[c, pl.ds(off, CH)])

    return k(dst_i, zb, ones_h)


def _dot(a, b):
    return jnp.dot(a, b, preferred_element_type=jnp.float32)


def _tc_sage(x, acc, deg, wself, wneigh, b, residual):
    def body(x_r, acc_r, deg_r, ws_r, wn_r, b_r, out_r):
        xv = x_r[...]
        av = acc_r[...]
        dv = deg_r[...]
        ws = ws_r[...]
        wn = wn_r[...]
        bv = b_r[...]
        outs = []
        if xv.ndim == 2:
            fs = _dot(xv, ws)
            for g in range(2):
                hn = av[g] / jnp.maximum(dv[g, :, 0:1], 1.0)
                outs.append(fs + _dot(hn, wn) + bv)
        else:
            for g in range(2):
                hn = av[g] / jnp.maximum(dv[g, :, 0:1], 1.0)
                o = _dot(xv[g], ws) + _dot(hn, wn) + bv
                if residual:
                    o = o + xv[g]
                outs.append(o)
        out_r[...] = jnp.stack(outs)

    x_spec = (pl.BlockSpec((BN, D), lambda i: (i, 0)) if x.ndim == 2
              else pl.BlockSpec((2, BN, D), lambda i: (0, i, 0)))
    return pl.pallas_call(
        body,
        grid=(G_BLK,),
        in_specs=[
            x_spec,
            pl.BlockSpec((2, BN, D), lambda i: (0, i, 0)),
            pl.BlockSpec((2, BN, 16), lambda i: (0, i, 0)),
            pl.BlockSpec((D, H), lambda i: (0, 0)),
            pl.BlockSpec((D, H), lambda i: (0, 0)),
            pl.BlockSpec((1, H), lambda i: (0, 0)),
        ],
        out_specs=pl.BlockSpec((2, BN, H), lambda i: (0, i, 0)),
        out_shape=jax.ShapeDtypeStruct((2, N, H), jnp.float32),
    )(x, acc, deg, wself, wneigh, b)


def _tc_attention(x2, w_omega, u_row):
    def body(x_r, wo_r, u_r, comb_r, alpha_r):
        xv = x_r[...]
        wo = wo_r[...]
        uv = u_r[...]
        v0 = jnp.tanh(_dot(xv[0], wo))
        v1 = jnp.tanh(_dot(xv[1], wo))
        vu0 = jnp.sum(v0 * uv, axis=1, keepdims=True) + 1e-6
        vu1 = jnp.sum(v1 * uv, axis=1, keepdims=True) + 1e-6
        m = jnp.maximum(vu0, vu1)
        e0 = jnp.exp(vu0 - m)
        e1 = jnp.exp(vu1 - m)
        tot = e0 + e1
        a0 = e0 / tot
        a1 = e1 / tot
        comb_r[...] = a0 * xv[0] + a1 * xv[1]
        alpha_r[...] = jnp.concatenate([a0, a1], axis=1)

    return pl.pallas_call(
        body,
        grid=(G_BLK,),
        in_specs=[
            pl.BlockSpec((2, BN, H), lambda i: (0, i, 0)),
            pl.BlockSpec((H, H), lambda i: (0, 0)),
            pl.BlockSpec((1, H), lambda i: (0, 0)),
        ],
        out_specs=[
            pl.BlockSpec((BN, H), lambda i: (i, 0)),
            pl.BlockSpec((BN, 2), lambda i: (i, 0)),
        ],
        out_shape=[
            jax.ShapeDtypeStruct((N, H), jnp.float32),
            jax.ShapeDtypeStruct((N, 2), jnp.float32),
        ],
    )(x2, w_omega, u_row)


def _tc_stats1(comb, w1, bl1):
    def body(c_r, w_r, b_r, st_r, acc):
        i = pl.program_id(0)

        @pl.when(i == 0)
        def _():
            acc[...] = jnp.zeros_like(acc)

        y = _dot(c_r[...], w_r[...]) + b_r[...]
        acc[0, :] += jnp.sum(y, axis=0)
        acc[1, :] += jnp.sum(y * y, axis=0)

        @pl.when(i == G_BLK - 1)
        def _():
            st_r[...] = acc[...]

    return pl.pallas_call(
        body,
        grid=(G_BLK,),
        in_specs=[
            pl.BlockSpec((BN, H), lambda i: (i, 0)),
            pl.BlockSpec((H, 1024), lambda i: (0, 0)),
            pl.BlockSpec((1, 1024), lambda i: (0, 0)),
        ],
        out_specs=pl.BlockSpec((2, 1024), lambda i: (0, 0)),
        out_shape=jax.ShapeDtypeStruct((2, 1024), jnp.float32),
        scratch_shapes=[pltpu.VMEM((2, 1024), jnp.float32)],
    )(comb, w1, bl1)


def _tc_mlp1(comb, st1, w1, bl1, g1, beta1, w2, bl2):
    def body(c_r, st_r, w1_r, b1_r, g1_r, be1_r, w2_r, b2_r, y2_r, st2_r, acc):
        i = pl.program_id(0)

        @pl.when(i == 0)
        def _():
            acc[...] = jnp.zeros_like(acc)

        st = st_r[...]
        mean = st[0:1, :] / N
        var = st[1:2, :] / N - mean * mean
        scale = g1_r[...] * lax.rsqrt(var + 1e-5)
        shift = be1_r[...] - mean * scale
        y1 = _dot(c_r[...], w1_r[...]) + b1_r[...]
        x = jnp.maximum(y1 * scale + shift, 0.0)
        y2 = _dot(x, w2_r[...]) + b2_r[...]
        y2_r[...] = y2
        acc[0, :] += jnp.sum(y2, axis=0)
        acc[1, :] += jnp.sum(y2 * y2, axis=0)

        @pl.when(i == G_BLK - 1)
        def _():
            st2_r[...] = acc[...]

    return pl.pallas_call(
        body,
        grid=(G_BLK,),
        in_specs=[
            pl.BlockSpec((BN, H), lambda i: (i, 0)),
            pl.BlockSpec((2, 1024), lambda i: (0, 0)),
            pl.BlockSpec((H, 1024), lambda i: (0, 0)),
            pl.BlockSpec((1, 1024), lambda i: (0, 0)),
            pl.BlockSpec((1, 1024), lambda i: (0, 0)),
            pl.BlockSpec((1, 1024), lambda i: (0, 0)),
            pl.BlockSpec((1024, H), lambda i: (0, 0)),
            pl.BlockSpec((1, H), lambda i: (0, 0)),
        ],
        out_specs=[
            pl.BlockSpec((BN, H), lambda i: (i, 0)),
            pl.BlockSpec((2, H), lambda i: (0, 0)),
        ],
        out_shape=[
            jax.ShapeDtypeStruct((N, H), jnp.float32),
            jax.ShapeDtypeStruct((2, H), jnp.float32),
        ],
        scratch_shapes=[pltpu.VMEM((2, H), jnp.float32)],
    )(comb, st1, w1, bl1, g1, beta1, w2, bl2)


def _tc_mlp2(y2, st2, g2, beta2, w3, bl3):
    def body(y_r, st_r, g_r, be_r, w_r, b_r, out_r):
        st = st_r[...]
        mean = st[0:1, :] / N
        var = st[1:2, :] / N - mean * mean
        scale = g_r[...] * lax.rsqrt(var + 1e-5)
        shift = be_r[...] - mean * scale
        x = jnp.maximum(y_r[...] * scale + shift, 0.0)
        out_r[...] = _dot(x, w_r[...]) + b_r[...]

    return pl.pallas_call(
        body,
        grid=(G_BLK,),
        in_specs=[
            pl.BlockSpec((BN, H), lambda i: (i, 0)),
            pl.BlockSpec((2, H), lambda i: (0, 0)),
            pl.BlockSpec((1, H), lambda i: (0, 0)),
            pl.BlockSpec((1, H), lambda i: (0, 0)),
            pl.BlockSpec((H, OUT), lambda i: (0, 0)),
            pl.BlockSpec((1, OUT), lambda i: (0, 0)),
        ],
        out_specs=pl.BlockSpec((BN, OUT), lambda i: (i, 0)),
        out_shape=jax.ShapeDtypeStruct((N, OUT), jnp.float32),
    )(y2, st2, g2, beta2, w3, bl3)


def _prep_edges(edge):
    src = jnp.concatenate([edge[0], jnp.zeros((E_PAD - E,), jnp.int32)])
    dst = jnp.concatenate([edge[1], jnp.full((E_PAD - E,), N, jnp.int32)])
    return src.reshape(NS * K, CH), dst.reshape(NS * K, CH)


def kernel(feat_omics1, edge_index_spatial, edge_index_feature, weight,
           Wself0, Wneigh0, b0, Wself1, Wneigh1, b1c, w_omega, u_omega,
           W1, bl1, g1, beta1, W2, bl2, g2, beta2, W3, bl3):
    zb = jnp.zeros((N_PAD, D), jnp.float32)
    ones_h = jnp.ones((CH, D), jnp.float32)

    src_s0, dst_s = _prep_edges(edge_index_spatial)
    src_f0, dst_f = _prep_edges(edge_index_feature)
    src_l0 = jnp.stack([src_s0, src_f0])
    src_l1 = jnp.stack([src_s0, src_f0 + N])
    dst_all = jnp.stack([dst_s, dst_f])

    b0r = b0.reshape(1, H)
    b1r = b1c.reshape(1, H)
    u_row = u_omega.reshape(1, H)
    bl1r = bl1.reshape(1, 1024)
    g1r = g1.reshape(1, 1024)
    beta1r = beta1.reshape(1, 1024)
    bl2r = bl2.reshape(1, H)
    g2r = g2.reshape(1, H)
    beta2r = beta2.reshape(1, H)
    bl3r = bl3.reshape(1, OUT)

    feat_pad = jnp.concatenate(
        [feat_omics1, jnp.zeros((N_PAD - N, D), jnp.float32)])
    deg = _sc_dual_deg(dst_all, zb, ones_h)
    acc0 = _sc_dual_segsum(feat_pad, src_l0, dst_all, zb)
    acc0 = acc0[:, :N, :]
    degN = deg[:, :N, :16]
    x1 = _tc_sage(feat_omics1, acc0, degN, Wself0, Wneigh0, b0r, False)

    table1 = x1.reshape(2 * N, H)
    acc1 = _sc_dual_segsum(table1, src_l1, dst_all, zb)
    acc1 = acc1[:, :N, :]
    x2 = _tc_sage(x1, acc1, degN, Wself1, Wneigh1, b1r, True)

    comb, alpha = _tc_attention(x2, w_omega, u_row)

    st1 = _tc_stats1(comb, W1, bl1r)
    y2, st2 = _tc_mlp1(comb, st1, W1, bl1r, g1r, beta1r, W2, bl2r)
    out = _tc_mlp2(y2, st2, g2r, beta2r, W3, bl3r)
    return (out, alpha)


# speedup vs baseline: 1.0566x; 1.0566x over previous
import functools

import jax
import jax.numpy as jnp
from jax import lax
from jax.experimental import pallas as pl
from jax.experimental.pallas import tpu as pltpu
from jax.experimental.pallas import tpu_sc as plsc

N = 10000
E = 320000
D = 128
H = 128
OUT = 64

NS = 16
CH = 128
K = 160
KI = 16
E_PAD = NS * K * CH
N_PAD = 10240
RPT = N_PAD // NS
PIECES = RPT // CH
BN = 1000
G_BLK = N // BN


def _sc_mesh():
    return plsc.VectorSubcoreMesh(core_axis_name="c", subcore_axis_name="s")


def _sc_dual_segsum(table, src_i, dst_i, zb):
    @functools.partial(
        pl.kernel,
        out_type=jax.ShapeDtypeStruct((2, N_PAD, D), jnp.float32),
        mesh=_sc_mesh(),
        scratch_types=[
            pltpu.VMEM((KI, CH), jnp.int32),
            pltpu.VMEM((KI, CH), jnp.int32),
            pltpu.VMEM((CH, D), jnp.float32),
            pltpu.VMEM((CH, D), jnp.float32),
            pltpu.SemaphoreType.DMA,
            pltpu.SemaphoreType.DMA,
            pltpu.VMEM_SHARED((N_PAD, D), jnp.float32),
        ],
    )
    def k(table_h, src_h, dst_h, zb_h, out_h,
          src_v, dst_v, rows_a, rows_b, sem_a, sem_b, acc_sh):
        c = lax.axis_index("c")
        s = lax.axis_index("s")
        rs = s * RPT

        @pl.loop(0, PIECES)
        def _(p):
            off = rs + p * CH
            pltpu.sync_copy(zb_h.at[pl.ds(off, CH)], acc_sh.at[pl.ds(off, CH)])

        plsc.subcore_barrier()

        @pl.loop(0, K // KI)
        def _(o):
            pltpu.sync_copy(src_h.at[c, pl.ds(s * K + o * KI, KI)], src_v)
            pltpu.sync_copy(dst_h.at[c, pl.ds(s * K + o * KI, KI)], dst_v)
            pltpu.async_copy(table_h.at[src_v.at[0]], rows_a, sem_a)

            @pl.loop(0, KI // 2)
            def _(jj):
                j0 = 2 * jj
                pltpu.make_async_copy(table_h.at[src_v.at[j0]], rows_a, sem_a).wait()
                pltpu.async_copy(table_h.at[src_v.at[j0 + 1]], rows_b, sem_b)
                pltpu.sync_copy(rows_a, acc_sh.at[dst_v.at[j0]], add=True)
                pltpu.make_async_copy(table_h.at[src_v.at[j0 + 1]], rows_b, sem_b).wait()

                @pl.when(jj < KI // 2 - 1)
                def _():
                    pltpu.async_copy(table_h.at[src_v.at[j0 + 2]], rows_a, sem_a)

                pltpu.sync_copy(rows_b, acc_sh.at[dst_v.at[j0 + 1]], add=True)

        plsc.subcore_barrier()

        @pl.loop(0, PIECES)
        def _(p):
            off = rs + p * CH
            pltpu.sync_copy(acc_sh.at[pl.ds(off, CH)], out_h.at[c, pl.ds(off, CH)])

    return k(table, src_i, dst_i, zb)


def _sc_dual_deg(dst_i, zb, ones_h):
    @functools.partial(
        pl.kernel,
        out_type=jax.ShapeDtypeStruct((2, N_PAD, D), jnp.float32),
        mesh=_sc_mesh(),
        scratch_types=[
            pltpu.VMEM((KI, CH), jnp.int32),
            pltpu.VMEM((CH, D), jnp.float32),
            pltpu.VMEM_SHARED((N_PAD, D), jnp.float32),
        ],
    )
    def k(dst_h, zb_h, ones_hh, out_h, dst_v, ones_v, deg_sh):
        c = lax.axis_index("c")
        s = lax.axis_index("s")
        rs = s * RPT

        @pl.loop(0, PIECES)
        def _(p):
            off = rs + p * CH
            pltpu.sync_copy(zb_h.at[pl.ds(off, CH)], deg_sh.at[pl.ds(off, CH)])

        pltpu.sync_copy(ones_hh, ones_v)
        plsc.subcore_barrier()

        @pl.loop(0, K // KI)
        def _(o):
            pltpu.sync_copy(dst_h.at[c, pl.ds(s * K + o * KI, KI)], dst_v)

            @pl.loop(0, KI)
            def _(j):
                pltpu.sync_copy(ones_v, deg_sh.at[dst_v.at[j]], add=True)

        plsc.subcore_barrier()

        @pl.loop(0, PIECES)
        def _(p):
            off = rs + p * CH
            pltpu.sync_copy(deg_sh.at[pl.ds(off, CH)], out_h.at[c, pl.ds(off, CH)])

    return k(dst_i, zb, ones_h)


def _dot(a, b):
    return jnp.dot(a, b, preferred_element_type=jnp.float32)


def _tc_sage(x, acc, deg, wself, wneigh, b, residual):
    def body(x_r, acc_r, deg_r, ws_r, wn_r, b_r, out_r):
        xv = x_r[...]
        av = acc_r[...]
        dv = deg_r[...]
        ws = ws_r[...]
        wn = wn_r[...]
        bv = b_r[...]
        outs = []
        if xv.ndim == 2:
            fs = _dot(xv, ws)
            for g in range(2):
                hn = av[g] / jnp.maximum(dv[g, :, 0:1], 1.0)
                outs.append(fs + _dot(hn, wn) + bv)
        else:
            for g in range(2):
                hn = av[g] / jnp.maximum(dv[g, :, 0:1], 1.0)
                o = _dot(xv[g], ws) + _dot(hn, wn) + bv
                if residual:
                    o = o + xv[g]
                outs.append(o)
        out_r[...] = jnp.stack(outs)

    x_spec = (pl.BlockSpec((BN, D), lambda i: (i, 0)) if x.ndim == 2
              else pl.BlockSpec((2, BN, D), lambda i: (0, i, 0)))
    return pl.pallas_call(
        body,
        grid=(G_BLK,),
        in_specs=[
            x_spec,
            pl.BlockSpec((2, BN, D), lambda i: (0, i, 0)),
            pl.BlockSpec((2, BN, 16), lambda i: (0, i, 0)),
            pl.BlockSpec((D, H), lambda i: (0, 0)),
            pl.BlockSpec((D, H), lambda i: (0, 0)),
            pl.BlockSpec((1, H), lambda i: (0, 0)),
        ],
        out_specs=pl.BlockSpec((2, BN, H), lambda i: (0, i, 0)),
        out_shape=jax.ShapeDtypeStruct((2, N, H), jnp.float32),
    )(x, acc, deg, wself, wneigh, b)


def _tc_attention(x2, w_omega, u_row):
    def body(x_r, wo_r, u_r, comb_r, alpha_r):
        xv = x_r[...]
        wo = wo_r[...]
        uv = u_r[...]
        v0 = jnp.tanh(_dot(xv[0], wo))
        v1 = jnp.tanh(_dot(xv[1], wo))
        vu0 = jnp.sum(v0 * uv, axis=1, keepdims=True) + 1e-6
        vu1 = jnp.sum(v1 * uv, axis=1, keepdims=True) + 1e-6
        m = jnp.maximum(vu0, vu1)
        e0 = jnp.exp(vu0 - m)
        e1 = jnp.exp(vu1 - m)
        tot = e0 + e1
        a0 = e0 / tot
        a1 = e1 / tot
        comb_r[...] = a0 * xv[0] + a1 * xv[1]
        alpha_r[...] = jnp.concatenate([a0, a1], axis=1)

    return pl.pallas_call(
        body,
        grid=(G_BLK,),
        in_specs=[
            pl.BlockSpec((2, BN, H), lambda i: (0, i, 0)),
            pl.BlockSpec((H, H), lambda i: (0, 0)),
            pl.BlockSpec((1, H), lambda i: (0, 0)),
        ],
        out_specs=[
            pl.BlockSpec((BN, H), lambda i: (i, 0)),
            pl.BlockSpec((BN, 2), lambda i: (i, 0)),
        ],
        out_shape=[
            jax.ShapeDtypeStruct((N, H), jnp.float32),
            jax.ShapeDtypeStruct((N, 2), jnp.float32),
        ],
    )(x2, w_omega, u_row)


def _tc_stats1(comb, w1, bl1):
    def body(c_r, w_r, b_r, st_r, acc):
        i = pl.program_id(0)

        @pl.when(i == 0)
        def _():
            acc[...] = jnp.zeros_like(acc)

        y = _dot(c_r[...], w_r[...]) + b_r[...]
        acc[0, :] += jnp.sum(y, axis=0)
        acc[1, :] += jnp.sum(y * y, axis=0)

        @pl.when(i == G_BLK - 1)
        def _():
            st_r[...] = acc[...]

    return pl.pallas_call(
        body,
        grid=(G_BLK,),
        in_specs=[
            pl.BlockSpec((BN, H), lambda i: (i, 0)),
            pl.BlockSpec((H, 1024), lambda i: (0, 0)),
            pl.BlockSpec((1, 1024), lambda i: (0, 0)),
        ],
        out_specs=pl.BlockSpec((2, 1024), lambda i: (0, 0)),
        out_shape=jax.ShapeDtypeStruct((2, 1024), jnp.float32),
        scratch_shapes=[pltpu.VMEM((2, 1024), jnp.float32)],
    )(comb, w1, bl1)


def _tc_mlp1(comb, st1, w1, bl1, g1, beta1, w2, bl2):
    def body(c_r, st_r, w1_r, b1_r, g1_r, be1_r, w2_r, b2_r, y2_r, st2_r, acc):
        i = pl.program_id(0)

        @pl.when(i == 0)
        def _():
            acc[...] = jnp.zeros_like(acc)

        st = st_r[...]
        mean = st[0:1, :] / N
        var = st[1:2, :] / N - mean * mean
        scale = g1_r[...] * lax.rsqrt(var + 1e-5)
        shift = be1_r[...] - mean * scale
        y1 = _dot(c_r[...], w1_r[...]) + b1_r[...]
        x = jnp.maximum(y1 * scale + shift, 0.0)
        y2 = _dot(x, w2_r[...]) + b2_r[...]
        y2_r[...] = y2
        acc[0, :] += jnp.sum(y2, axis=0)
        acc[1, :] += jnp.sum(y2 * y2, axis=0)

        @pl.when(i == G_BLK - 1)
        def _():
            st2_r[...] = acc[...]

    return pl.pallas_call(
        body,
        grid=(G_BLK,),
        in_specs=[
            pl.BlockSpec((BN, H), lambda i: (i, 0)),
            pl.BlockSpec((2, 1024), lambda i: (0, 0)),
            pl.BlockSpec((H, 1024), lambda i: (0, 0)),
            pl.BlockSpec((1, 1024), lambda i: (0, 0)),
            pl.BlockSpec((1, 1024), lambda i: (0, 0)),
            pl.BlockSpec((1, 1024), lambda i: (0, 0)),
            pl.BlockSpec((1024, H), lambda i: (0, 0)),
            pl.BlockSpec((1, H), lambda i: (0, 0)),
        ],
        out_specs=[
            pl.BlockSpec((BN, H), lambda i: (i, 0)),
            pl.BlockSpec((2, H), lambda i: (0, 0)),
        ],
        out_shape=[
            jax.ShapeDtypeStruct((N, H), jnp.float32),
            jax.ShapeDtypeStruct((2, H), jnp.float32),
        ],
        scratch_shapes=[pltpu.VMEM((2, H), jnp.float32)],
    )(comb, st1, w1, bl1, g1, beta1, w2, bl2)


def _tc_mlp2(y2, st2, g2, beta2, w3, bl3):
    def body(y_r, st_r, g_r, be_r, w_r, b_r, out_r):
        st = st_r[...]
        mean = st[0:1, :] / N
        var = st[1:2, :] / N - mean * mean
        scale = g_r[...] * lax.rsqrt(var + 1e-5)
        shift = be_r[...] - mean * scale
        x = jnp.maximum(y_r[...] * scale + shift, 0.0)
        out_r[...] = _dot(x, w_r[...]) + b_r[...]

    return pl.pallas_call(
        body,
        grid=(G_BLK,),
        in_specs=[
            pl.BlockSpec((BN, H), lambda i: (i, 0)),
            pl.BlockSpec((2, H), lambda i: (0, 0)),
            pl.BlockSpec((1, H), lambda i: (0, 0)),
            pl.BlockSpec((1, H), lambda i: (0, 0)),
            pl.BlockSpec((H, OUT), lambda i: (0, 0)),
            pl.BlockSpec((1, OUT), lambda i: (0, 0)),
        ],
        out_specs=pl.BlockSpec((BN, OUT), lambda i: (i, 0)),
        out_shape=jax.ShapeDtypeStruct((N, OUT), jnp.float32),
    )(y2, st2, g2, beta2, w3, bl3)


def _prep_edges(edge):
    src = jnp.concatenate([edge[0], jnp.zeros((E_PAD - E,), jnp.int32)])
    dst = jnp.concatenate([edge[1], jnp.full((E_PAD - E,), N, jnp.int32)])
    return src.reshape(NS * K, CH), dst.reshape(NS * K, CH)


def kernel(feat_omics1, edge_index_spatial, edge_index_feature, weight,
           Wself0, Wneigh0, b0, Wself1, Wneigh1, b1c, w_omega, u_omega,
           W1, bl1, g1, beta1, W2, bl2, g2, beta2, W3, bl3):
    zb = jnp.zeros((N_PAD, D), jnp.float32)
    ones_h = jnp.ones((CH, D), jnp.float32)

    src_s0, dst_s = _prep_edges(edge_index_spatial)
    src_f0, dst_f = _prep_edges(edge_index_feature)
    src_l0 = jnp.stack([src_s0, src_f0])
    src_l1 = jnp.stack([src_s0, src_f0 + N])
    dst_all = jnp.stack([dst_s, dst_f])

    b0r = b0.reshape(1, H)
    b1r = b1c.reshape(1, H)
    u_row = u_omega.reshape(1, H)
    bl1r = bl1.reshape(1, 1024)
    g1r = g1.reshape(1, 1024)
    beta1r = beta1.reshape(1, 1024)
    bl2r = bl2.reshape(1, H)
    g2r = g2.reshape(1, H)
    beta2r = beta2.reshape(1, H)
    bl3r = bl3.reshape(1, OUT)

    feat2 = jnp.concatenate([feat_omics1, feat_omics1])
    deg = _sc_dual_deg(dst_all, zb, ones_h)
    acc0 = _sc_dual_segsum(feat2, src_l1, dst_all, zb)
    acc0 = acc0[:, :N, :]
    degN = deg[:, :N, :16]
    x1 = _tc_sage(feat_omics1, acc0, degN, Wself0, Wneigh0, b0r, False)

    table1 = x1.reshape(2 * N, H)
    acc1 = _sc_dual_segsum(table1, src_l1, dst_all, zb)
    acc1 = acc1[:, :N, :]
    x2 = _tc_sage(x1, acc1, degN, Wself1, Wneigh1, b1r, True)

    comb, alpha = _tc_attention(x2, w_omega, u_row)

    st1 = _tc_stats1(comb, W1, bl1r)
    y2, st2 = _tc_mlp1(comb, st1, W1, bl1r, g1r, beta1r, W2, bl2r)
    out = _tc_mlp2(y2, st2, g2r, beta2r, W3, bl3r)
    return (out, alpha)


# speedup vs baseline: 1.0688x; 1.0115x over previous
import functools

import jax
import jax.numpy as jnp
from jax import lax
from jax.experimental import pallas as pl
from jax.experimental.pallas import tpu as pltpu
from jax.experimental.pallas import tpu_sc as plsc

N = 10000
E = 320000
D = 128
H = 128
OUT = 64

NS = 16
CH = 128
K = 160
KI = 16
E_PAD = NS * K * CH
N_PAD = 10240
RPT = N_PAD // NS
PIECES = RPT // CH
BN = 1000
G_BLK = N // BN


def _sc_mesh():
    return plsc.VectorSubcoreMesh(core_axis_name="c", subcore_axis_name="s")


def _sc_dual_segsum(table, src_i, dst_i, zb):
    @functools.partial(
        pl.kernel,
        out_type=jax.ShapeDtypeStruct((2, N_PAD, D), jnp.float32),
        mesh=_sc_mesh(),
        scratch_types=[
            pltpu.VMEM((KI, CH), jnp.int32),
            pltpu.VMEM((KI, CH), jnp.int32),
            pltpu.VMEM((CH, D), jnp.float32),
            pltpu.VMEM((CH, D), jnp.float32),
            pltpu.SemaphoreType.DMA,
            pltpu.SemaphoreType.DMA,
            pltpu.VMEM_SHARED((N_PAD, D), jnp.float32),
        ],
    )
    def k(table_h, src_h, dst_h, zb_h, out_h,
          src_v, dst_v, rows_a, rows_b, sem_a, sem_b, acc_sh):
        c = lax.axis_index("c")
        s = lax.axis_index("s")
        rs = s * RPT

        @pl.loop(0, PIECES)
        def _(p):
            off = rs + p * CH
            pltpu.sync_copy(zb_h.at[pl.ds(off, CH)], acc_sh.at[pl.ds(off, CH)])

        plsc.subcore_barrier()

        @pl.loop(0, K // KI)
        def _(o):
            pltpu.sync_copy(src_h.at[c, pl.ds(s * K + o * KI, KI)], src_v)
            pltpu.sync_copy(dst_h.at[c, pl.ds(s * K + o * KI, KI)], dst_v)
            pltpu.async_copy(table_h.at[src_v.at[0]], rows_a, sem_a)

            @pl.loop(0, KI // 2)
            def _(jj):
                j0 = 2 * jj
                pltpu.make_async_copy(table_h.at[src_v.at[j0]], rows_a, sem_a).wait()
                pltpu.async_copy(table_h.at[src_v.at[j0 + 1]], rows_b, sem_b)
                pltpu.sync_copy(rows_a, acc_sh.at[dst_v.at[j0]], add=True)
                pltpu.make_async_copy(table_h.at[src_v.at[j0 + 1]], rows_b, sem_b).wait()

                @pl.when(jj < KI // 2 - 1)
                def _():
                    pltpu.async_copy(table_h.at[src_v.at[j0 + 2]], rows_a, sem_a)

                pltpu.sync_copy(rows_b, acc_sh.at[dst_v.at[j0 + 1]], add=True)

        plsc.subcore_barrier()

        @pl.loop(0, PIECES)
        def _(p):
            off = rs + p * CH
            pltpu.sync_copy(acc_sh.at[pl.ds(off, CH)], out_h.at[c, pl.ds(off, CH)])

    return k(table, src_i, dst_i, zb)


def _sc_dual_deg(dst_i, zb, ones_h):
    @functools.partial(
        pl.kernel,
        out_type=jax.ShapeDtypeStruct((2, N_PAD, D), jnp.float32),
        mesh=_sc_mesh(),
        scratch_types=[
            pltpu.VMEM((KI, CH), jnp.int32),
            pltpu.VMEM((CH, D), jnp.float32),
            pltpu.VMEM_SHARED((N_PAD, D), jnp.float32),
        ],
    )
    def k(dst_h, zb_h, ones_hh, out_h, dst_v, ones_v, deg_sh):
        c = lax.axis_index("c")
        s = lax.axis_index("s")
        rs = s * RPT

        @pl.loop(0, PIECES)
        def _(p):
            off = rs + p * CH
            pltpu.sync_copy(zb_h.at[pl.ds(off, CH)], deg_sh.at[pl.ds(off, CH)])

        pltpu.sync_copy(ones_hh, ones_v)
        plsc.subcore_barrier()

        @pl.loop(0, K // KI)
        def _(o):
            pltpu.sync_copy(dst_h.at[c, pl.ds(s * K + o * KI, KI)], dst_v)

            @pl.loop(0, KI)
            def _(j):
                pltpu.sync_copy(ones_v, deg_sh.at[dst_v.at[j]], add=True)

        plsc.subcore_barrier()

        @pl.loop(0, PIECES)
        def _(p):
            off = rs + p * CH
            pltpu.sync_copy(deg_sh.at[pl.ds(off, CH)], out_h.at[c, pl.ds(off, CH)])

    return k(dst_i, zb, ones_h)


def _dot(a, b):
    return jnp.dot(a, b, preferred_element_type=jnp.float32)


def _tc_sage(x, acc, deg, wself, wneigh, b, residual):
    def body(x_r, acc_r, deg_r, ws_r, wn_r, b_r, out_r):
        xv = x_r[...]
        av = acc_r[...]
        dv = deg_r[...]
        ws = ws_r[...]
        wn = wn_r[...]
        bv = b_r[...]
        outs = []
        if xv.ndim == 2:
            fs = _dot(xv, ws)
            for g in range(2):
                hn = av[g] / jnp.maximum(dv[g, :, 0:1], 1.0)
                outs.append(fs + _dot(hn, wn) + bv)
        else:
            for g in range(2):
                hn = av[g] / jnp.maximum(dv[g, :, 0:1], 1.0)
                o = _dot(xv[g], ws) + _dot(hn, wn) + bv
                if residual:
                    o = o + xv[g]
                outs.append(o)
        out_r[...] = jnp.stack(outs)

    x_spec = (pl.BlockSpec((BN, D), lambda i: (i, 0)) if x.ndim == 2
              else pl.BlockSpec((2, BN, D), lambda i: (0, i, 0)))
    return pl.pallas_call(
        body,
        grid=(G_BLK,),
        in_specs=[
            x_spec,
            pl.BlockSpec((2, BN, D), lambda i: (0, i, 0)),
            pl.BlockSpec((2, BN, 16), lambda i: (0, i, 0)),
            pl.BlockSpec((D, H), lambda i: (0, 0)),
            pl.BlockSpec((D, H), lambda i: (0, 0)),
            pl.BlockSpec((1, H), lambda i: (0, 0)),
        ],
        out_specs=pl.BlockSpec((2, BN, H), lambda i: (0, i, 0)),
        out_shape=jax.ShapeDtypeStruct((2, N, H), jnp.float32),
    )(x, acc, deg, wself, wneigh, b)


def _tc_sage1_att(x1, acc1, deg, wself, wneigh, b, w_omega, u_row, w1, bl1):
    def body(x_r, acc_r, deg_r, ws_r, wn_r, b_r, wo_r, u_r, w1_r, b1_r,
             comb_r, alpha_r, st_r, accsc):
        i = pl.program_id(0)

        @pl.when(i == 0)
        def _():
            accsc[...] = jnp.zeros_like(accsc)

        xv = x_r[...]
        av = acc_r[...]
        dv = deg_r[...]
        ws = ws_r[...]
        wn = wn_r[...]
        bv = b_r[...]
        x2 = []
        for g in range(2):
            hn = av[g] / jnp.maximum(dv[g, :, 0:1], 1.0)
            x2.append(_dot(xv[g], ws) + _dot(hn, wn) + bv + xv[g])
        wo = wo_r[...]
        uv = u_r[...]
        v0 = jnp.tanh(_dot(x2[0], wo))
        v1 = jnp.tanh(_dot(x2[1], wo))
        vu0 = jnp.sum(v0 * uv, axis=1, keepdims=True) + 1e-6
        vu1 = jnp.sum(v1 * uv, axis=1, keepdims=True) + 1e-6
        m = jnp.maximum(vu0, vu1)
        e0 = jnp.exp(vu0 - m)
        e1 = jnp.exp(vu1 - m)
        tot = e0 + e1
        a0 = e0 / tot
        a1 = e1 / tot
        comb = a0 * x2[0] + a1 * x2[1]
        comb_r[...] = comb
        alpha_r[...] = jnp.concatenate([a0, a1], axis=1)
        y = _dot(comb, w1_r[...]) + b1_r[...]
        accsc[0, :] += jnp.sum(y, axis=0)
        accsc[1, :] += jnp.sum(y * y, axis=0)

        @pl.when(i == G_BLK - 1)
        def _():
            st_r[...] = accsc[...]

    return pl.pallas_call(
        body,
        grid=(G_BLK,),
        in_specs=[
            pl.BlockSpec((2, BN, H), lambda i: (0, i, 0)),
            pl.BlockSpec((2, BN, H), lambda i: (0, i, 0)),
            pl.BlockSpec((2, BN, 16), lambda i: (0, i, 0)),
            pl.BlockSpec((H, H), lambda i: (0, 0)),
            pl.BlockSpec((H, H), lambda i: (0, 0)),
            pl.BlockSpec((1, H), lambda i: (0, 0)),
            pl.BlockSpec((H, H), lambda i: (0, 0)),
            pl.BlockSpec((1, H), lambda i: (0, 0)),
            pl.BlockSpec((H, 1024), lambda i: (0, 0)),
            pl.BlockSpec((1, 1024), lambda i: (0, 0)),
        ],
        out_specs=[
            pl.BlockSpec((BN, H), lambda i: (i, 0)),
            pl.BlockSpec((BN, 2), lambda i: (i, 0)),
            pl.BlockSpec((2, 1024), lambda i: (0, 0)),
        ],
        out_shape=[
            jax.ShapeDtypeStruct((N, H), jnp.float32),
            jax.ShapeDtypeStruct((N, 2), jnp.float32),
            jax.ShapeDtypeStruct((2, 1024), jnp.float32),
        ],
        scratch_shapes=[pltpu.VMEM((2, 1024), jnp.float32)],
    )(x1, acc1, deg, wself, wneigh, b, w_omega, u_row, w1, bl1)


def _tc_attention(x2, w_omega, u_row):
    def body(x_r, wo_r, u_r, comb_r, alpha_r):
        xv = x_r[...]
        wo = wo_r[...]
        uv = u_r[...]
        v0 = jnp.tanh(_dot(xv[0], wo))
        v1 = jnp.tanh(_dot(xv[1], wo))
        vu0 = jnp.sum(v0 * uv, axis=1, keepdims=True) + 1e-6
        vu1 = jnp.sum(v1 * uv, axis=1, keepdims=True) + 1e-6
        m = jnp.maximum(vu0, vu1)
        e0 = jnp.exp(vu0 - m)
        e1 = jnp.exp(vu1 - m)
        tot = e0 + e1
        a0 = e0 / tot
        a1 = e1 / tot
        comb_r[...] = a0 * xv[0] + a1 * xv[1]
        alpha_r[...] = jnp.concatenate([a0, a1], axis=1)

    return pl.pallas_call(
        body,
        grid=(G_BLK,),
        in_specs=[
            pl.BlockSpec((2, BN, H), lambda i: (0, i, 0)),
            pl.BlockSpec((H, H), lambda i: (0, 0)),
            pl.BlockSpec((1, H), lambda i: (0, 0)),
        ],
        out_specs=[
            pl.BlockSpec((BN, H), lambda i: (i, 0)),
            pl.BlockSpec((BN, 2), lambda i: (i, 0)),
        ],
        out_shape=[
            jax.ShapeDtypeStruct((N, H), jnp.float32),
            jax.ShapeDtypeStruct((N, 2), jnp.float32),
        ],
    )(x2, w_omega, u_row)


def _tc_stats1(comb, w1, bl1):
    def body(c_r, w_r, b_r, st_r, acc):
        i = pl.program_id(0)

        @pl.when(i == 0)
        def _():
            acc[...] = jnp.zeros_like(acc)

        y = _dot(c_r[...], w_r[...]) + b_r[...]
        acc[0, :] += jnp.sum(y, axis=0)
        acc[1, :] += jnp.sum(y * y, axis=0)

        @pl.when(i == G_BLK - 1)
        def _():
            st_r[...] = acc[...]

    return pl.pallas_call(
        body,
        grid=(G_BLK,),
        in_specs=[
            pl.BlockSpec((BN, H), lambda i: (i, 0)),
            pl.BlockSpec((H, 1024), lambda i: (0, 0)),
            pl.BlockSpec((1, 1024), lambda i: (0, 0)),
        ],
        out_specs=pl.BlockSpec((2, 1024), lambda i: (0, 0)),
        out_shape=jax.ShapeDtypeStruct((2, 1024), jnp.float32),
        scratch_shapes=[pltpu.VMEM((2, 1024), jnp.float32)],
    )(comb, w1, bl1)


def _tc_mlp1(comb, st1, w1, bl1, g1, beta1, w2, bl2):
    def body(c_r, st_r, w1_r, b1_r, g1_r, be1_r, w2_r, b2_r, y2_r, st2_r, acc):
        i = pl.program_id(0)

        @pl.when(i == 0)
        def _():
            acc[...] = jnp.zeros_like(acc)

        st = st_r[...]
        mean = st[0:1, :] / N
        var = st[1:2, :] / N - mean * mean
        scale = g1_r[...] * lax.rsqrt(var + 1e-5)
        shift = be1_r[...] - mean * scale
        y1 = _dot(c_r[...], w1_r[...]) + b1_r[...]
        x = jnp.maximum(y1 * scale + shift, 0.0)
        y2 = _dot(x, w2_r[...]) + b2_r[...]
        y2_r[...] = y2
        acc[0, :] += jnp.sum(y2, axis=0)
        acc[1, :] += jnp.sum(y2 * y2, axis=0)

        @pl.when(i == G_BLK - 1)
        def _():
            st2_r[...] = acc[...]

    return pl.pallas_call(
        body,
        grid=(G_BLK,),
        in_specs=[
            pl.BlockSpec((BN, H), lambda i: (i, 0)),
            pl.BlockSpec((2, 1024), lambda i: (0, 0)),
            pl.BlockSpec((H, 1024), lambda i: (0, 0)),
            pl.BlockSpec((1, 1024), lambda i: (0, 0)),
            pl.BlockSpec((1, 1024), lambda i: (0, 0)),
            pl.BlockSpec((1, 1024), lambda i: (0, 0)),
            pl.BlockSpec((1024, H), lambda i: (0, 0)),
            pl.BlockSpec((1, H), lambda i: (0, 0)),
        ],
        out_specs=[
            pl.BlockSpec((BN, H), lambda i: (i, 0)),
            pl.BlockSpec((2, H), lambda i: (0, 0)),
        ],
        out_shape=[
            jax.ShapeDtypeStruct((N, H), jnp.float32),
            jax.ShapeDtypeStruct((2, H), jnp.float32),
        ],
        scratch_shapes=[pltpu.VMEM((2, H), jnp.float32)],
    )(comb, st1, w1, bl1, g1, beta1, w2, bl2)


def _tc_mlp2(y2, st2, g2, beta2, w3, bl3):
    def body(y_r, st_r, g_r, be_r, w_r, b_r, out_r):
        st = st_r[...]
        mean = st[0:1, :] / N
        var = st[1:2, :] / N - mean * mean
        scale = g_r[...] * lax.rsqrt(var + 1e-5)
        shift = be_r[...] - mean * scale
        x = jnp.maximum(y_r[...] * scale + shift, 0.0)
        out_r[...] = _dot(x, w_r[...]) + b_r[...]

    return pl.pallas_call(
        body,
        grid=(G_BLK,),
        in_specs=[
            pl.BlockSpec((BN, H), lambda i: (i, 0)),
            pl.BlockSpec((2, H), lambda i: (0, 0)),
            pl.BlockSpec((1, H), lambda i: (0, 0)),
            pl.BlockSpec((1, H), lambda i: (0, 0)),
            pl.BlockSpec((H, OUT), lambda i: (0, 0)),
            pl.BlockSpec((1, OUT), lambda i: (0, 0)),
        ],
        out_specs=pl.BlockSpec((BN, OUT), lambda i: (i, 0)),
        out_shape=jax.ShapeDtypeStruct((N, OUT), jnp.float32),
    )(y2, st2, g2, beta2, w3, bl3)


def _prep_edges(edge):
    src = jnp.concatenate([edge[0], jnp.zeros((E_PAD - E,), jnp.int32)])
    dst = jnp.concatenate([edge[1], jnp.full((E_PAD - E,), N, jnp.int32)])
    return src.reshape(NS * K, CH), dst.reshape(NS * K, CH)


def kernel(feat_omics1, edge_index_spatial, edge_index_feature, weight,
           Wself0, Wneigh0, b0, Wself1, Wneigh1, b1c, w_omega, u_omega,
           W1, bl1, g1, beta1, W2, bl2, g2, beta2, W3, bl3):
    zb = jnp.zeros((N_PAD, D), jnp.float32)
    ones_h = jnp.ones((CH, D), jnp.float32)

    src_s0, dst_s = _prep_edges(edge_index_spatial)
    src_f0, dst_f = _prep_edges(edge_index_feature)
    src_l0 = jnp.stack([src_s0, src_f0])
    src_l1 = jnp.stack([src_s0, src_f0 + N])
    dst_all = jnp.stack([dst_s, dst_f])

    b0r = b0.reshape(1, H)
    b1r = b1c.reshape(1, H)
    u_row = u_omega.reshape(1, H)
    bl1r = bl1.reshape(1, 1024)
    g1r = g1.reshape(1, 1024)
    beta1r = beta1.reshape(1, 1024)
    bl2r = bl2.reshape(1, H)
    g2r = g2.reshape(1, H)
    beta2r = beta2.reshape(1, H)
    bl3r = bl3.reshape(1, OUT)

    feat2 = jnp.concatenate([feat_omics1, feat_omics1])
    deg = _sc_dual_deg(dst_all, zb, ones_h)
    acc0 = _sc_dual_segsum(feat2, src_l1, dst_all, zb)
    acc0 = acc0[:, :N, :]
    degN = deg[:, :N, :16]
    x1 = _tc_sage(feat_omics1, acc0, degN, Wself0, Wneigh0, b0r, False)

    table1 = x1.reshape(2 * N, H)
    acc1 = _sc_dual_segsum(table1, src_l1, dst_all, zb)
    acc1 = acc1[:, :N, :]
    comb, alpha, st1 = _tc_sage1_att(x1, acc1, degN, Wself1, Wneigh1, b1r,
                                     w_omega, u_row, W1, bl1r)
    y2, st2 = _tc_mlp1(comb, st1, W1, bl1r, g1r, beta1r, W2, bl2r)
    out = _tc_mlp2(y2, st2, g2r, beta2r, W3, bl3r)
    return (out, alpha)


# speedup vs baseline: 1.0829x; 1.0132x over previous
import functools

import jax
import jax.numpy as jnp
from jax import lax
from jax.experimental import pallas as pl
from jax.experimental.pallas import tpu as pltpu
from jax.experimental.pallas import tpu_sc as plsc

N = 10000
E = 320000
D = 128
H = 128
OUT = 64

NS = 16
CH = 128
K = 160
KI = 32
E_PAD = NS * K * CH
N_PAD = 10240
RPT = N_PAD // NS
PIECES = RPT // CH
BN = 1000
G_BLK = N // BN


def _sc_mesh():
    return plsc.VectorSubcoreMesh(core_axis_name="c", subcore_axis_name="s")


def _sc_dual_segsum(table, src_i, dst_i, zb):
    @functools.partial(
        pl.kernel,
        out_type=jax.ShapeDtypeStruct((2, N_PAD, D), jnp.float32),
        mesh=_sc_mesh(),
        scratch_types=[
            pltpu.VMEM((KI, CH), jnp.int32),
            pltpu.VMEM((KI, CH), jnp.int32),
            pltpu.VMEM((CH, D), jnp.float32),
            pltpu.VMEM((CH, D), jnp.float32),
            pltpu.SemaphoreType.DMA,
            pltpu.SemaphoreType.DMA,
            pltpu.VMEM_SHARED((N_PAD, D), jnp.float32),
        ],
    )
    def k(table_h, src_h, dst_h, zb_h, out_h,
          src_v, dst_v, rows_a, rows_b, sem_a, sem_b, acc_sh):
        c = lax.axis_index("c")
        s = lax.axis_index("s")
        rs = s * RPT

        @pl.loop(0, PIECES)
        def _(p):
            off = rs + p * CH
            pltpu.sync_copy(zb_h.at[pl.ds(off, CH)], acc_sh.at[pl.ds(off, CH)])

        plsc.subcore_barrier()

        @pl.loop(0, K // KI)
        def _(o):
            pltpu.sync_copy(src_h.at[c, pl.ds(s * K + o * KI, KI)], src_v)
            pltpu.sync_copy(dst_h.at[c, pl.ds(s * K + o * KI, KI)], dst_v)
            pltpu.async_copy(table_h.at[src_v.at[0]], rows_a, sem_a)

            @pl.loop(0, KI // 2)
            def _(jj):
                j0 = 2 * jj
                pltpu.make_async_copy(table_h.at[src_v.at[j0]], rows_a, sem_a).wait()
                pltpu.async_copy(table_h.at[src_v.at[j0 + 1]], rows_b, sem_b)
                pltpu.sync_copy(rows_a, acc_sh.at[dst_v.at[j0]], add=True)
                pltpu.make_async_copy(table_h.at[src_v.at[j0 + 1]], rows_b, sem_b).wait()

                @pl.when(jj < KI // 2 - 1)
                def _():
                    pltpu.async_copy(table_h.at[src_v.at[j0 + 2]], rows_a, sem_a)

                pltpu.sync_copy(rows_b, acc_sh.at[dst_v.at[j0 + 1]], add=True)

        plsc.subcore_barrier()

        @pl.loop(0, PIECES)
        def _(p):
            off = rs + p * CH
            pltpu.sync_copy(acc_sh.at[pl.ds(off, CH)], out_h.at[c, pl.ds(off, CH)])

    return k(table, src_i, dst_i, zb)


def _sc_dual_deg(dst_i, zb, ones_h):
    @functools.partial(
        pl.kernel,
        out_type=jax.ShapeDtypeStruct((2, N_PAD, D), jnp.float32),
        mesh=_sc_mesh(),
        scratch_types=[
            pltpu.VMEM((KI, CH), jnp.int32),
            pltpu.VMEM((CH, D), jnp.float32),
            pltpu.VMEM_SHARED((N_PAD, D), jnp.float32),
        ],
    )
    def k(dst_h, zb_h, ones_hh, out_h, dst_v, ones_v, deg_sh):
        c = lax.axis_index("c")
        s = lax.axis_index("s")
        rs = s * RPT

        @pl.loop(0, PIECES)
        def _(p):
            off = rs + p * CH
            pltpu.sync_copy(zb_h.at[pl.ds(off, CH)], deg_sh.at[pl.ds(off, CH)])

        pltpu.sync_copy(ones_hh, ones_v)
        plsc.subcore_barrier()

        @pl.loop(0, K // KI)
        def _(o):
            pltpu.sync_copy(dst_h.at[c, pl.ds(s * K + o * KI, KI)], dst_v)

            @pl.loop(0, KI)
            def _(j):
                pltpu.sync_copy(ones_v, deg_sh.at[dst_v.at[j]], add=True)

        plsc.subcore_barrier()

        @pl.loop(0, PIECES)
        def _(p):
            off = rs + p * CH
            pltpu.sync_copy(deg_sh.at[pl.ds(off, CH)], out_h.at[c, pl.ds(off, CH)])

    return k(dst_i, zb, ones_h)


def _dot(a, b):
    return jnp.dot(a, b, preferred_element_type=jnp.float32)


def _tc_sage(x, acc, deg, wself, wneigh, b, residual):
    def body(x_r, acc_r, deg_r, ws_r, wn_r, b_r, out_r):
        xv = x_r[...]
        av = acc_r[...]
        dv = deg_r[...]
        ws = ws_r[...]
        wn = wn_r[...]
        bv = b_r[...]
        outs = []
        if xv.ndim == 2:
            fs = _dot(xv, ws)
            for g in range(2):
                hn = av[g] / jnp.maximum(dv[g, :, 0:1], 1.0)
                outs.append(fs + _dot(hn, wn) + bv)
        else:
            for g in range(2):
                hn = av[g] / jnp.maximum(dv[g, :, 0:1], 1.0)
                o = _dot(xv[g], ws) + _dot(hn, wn) + bv
                if residual:
                    o = o + xv[g]
                outs.append(o)
        out_r[...] = jnp.stack(outs)

    x_spec = (pl.BlockSpec((BN, D), lambda i: (i, 0)) if x.ndim == 2
              else pl.BlockSpec((2, BN, D), lambda i: (0, i, 0)))
    return pl.pallas_call(
        body,
        grid=(G_BLK,),
        in_specs=[
            x_spec,
            pl.BlockSpec((2, BN, D), lambda i: (0, i, 0)),
            pl.BlockSpec((2, BN, 16), lambda i: (0, i, 0)),
            pl.BlockSpec((D, H), lambda i: (0, 0)),
            pl.BlockSpec((D, H), lambda i: (0, 0)),
            pl.BlockSpec((1, H), lambda i: (0, 0)),
        ],
        out_specs=pl.BlockSpec((2, BN, H), lambda i: (0, i, 0)),
        out_shape=jax.ShapeDtypeStruct((2, N, H), jnp.float32),
    )(x, acc, deg, wself, wneigh, b)


def _tc_sage1_att(x1, acc1, deg, wself, wneigh, b, w_omega, u_row, w1, bl1):
    def body(x_r, acc_r, deg_r, ws_r, wn_r, b_r, wo_r, u_r, w1_r, b1_r,
             comb_r, alpha_r, st_r, accsc):
        i = pl.program_id(0)

        @pl.when(i == 0)
        def _():
            accsc[...] = jnp.zeros_like(accsc)

        xv = x_r[...]
        av = acc_r[...]
        dv = deg_r[...]
        ws = ws_r[...]
        wn = wn_r[...]
        bv = b_r[...]
        x2 = []
        for g in range(2):
            hn = av[g] / jnp.maximum(dv[g, :, 0:1], 1.0)
            x2.append(_dot(xv[g], ws) + _dot(hn, wn) + bv + xv[g])
        wo = wo_r[...]
        uv = u_r[...]
        v0 = jnp.tanh(_dot(x2[0], wo))
        v1 = jnp.tanh(_dot(x2[1], wo))
        vu0 = jnp.sum(v0 * uv, axis=1, keepdims=True) + 1e-6
        vu1 = jnp.sum(v1 * uv, axis=1, keepdims=True) + 1e-6
        m = jnp.maximum(vu0, vu1)
        e0 = jnp.exp(vu0 - m)
        e1 = jnp.exp(vu1 - m)
        tot = e0 + e1
        a0 = e0 / tot
        a1 = e1 / tot
        comb = a0 * x2[0] + a1 * x2[1]
        comb_r[...] = comb
        alpha_r[...] = jnp.concatenate([a0, a1], axis=1)
        y = _dot(comb, w1_r[...]) + b1_r[...]
        accsc[0, :] += jnp.sum(y, axis=0)
        accsc[1, :] += jnp.sum(y * y, axis=0)

        @pl.when(i == G_BLK - 1)
        def _():
            st_r[...] = accsc[...]

    return pl.pallas_call(
        body,
        grid=(G_BLK,),
        in_specs=[
            pl.BlockSpec((2, BN, H), lambda i: (0, i, 0)),
            pl.BlockSpec((2, BN, H), lambda i: (0, i, 0)),
            pl.BlockSpec((2, BN, 16), lambda i: (0, i, 0)),
            pl.BlockSpec((H, H), lambda i: (0, 0)),
            pl.BlockSpec((H, H), lambda i: (0, 0)),
            pl.BlockSpec((1, H), lambda i: (0, 0)),
            pl.BlockSpec((H, H), lambda i: (0, 0)),
            pl.BlockSpec((1, H), lambda i: (0, 0)),
            pl.BlockSpec((H, 1024), lambda i: (0, 0)),
            pl.BlockSpec((1, 1024), lambda i: (0, 0)),
        ],
        out_specs=[
            pl.BlockSpec((BN, H), lambda i: (i, 0)),
            pl.BlockSpec((BN, 2), lambda i: (i, 0)),
            pl.BlockSpec((2, 1024), lambda i: (0, 0)),
        ],
        out_shape=[
            jax.ShapeDtypeStruct((N, H), jnp.float32),
            jax.ShapeDtypeStruct((N, 2), jnp.float32),
            jax.ShapeDtypeStruct((2, 1024), jnp.float32),
        ],
        scratch_shapes=[pltpu.VMEM((2, 1024), jnp.float32)],
    )(x1, acc1, deg, wself, wneigh, b, w_omega, u_row, w1, bl1)


def _tc_attention(x2, w_omega, u_row):
    def body(x_r, wo_r, u_r, comb_r, alpha_r):
        xv = x_r[...]
        wo = wo_r[...]
        uv = u_r[...]
        v0 = jnp.tanh(_dot(xv[0], wo))
        v1 = jnp.tanh(_dot(xv[1], wo))
        vu0 = jnp.sum(v0 * uv, axis=1, keepdims=True) + 1e-6
        vu1 = jnp.sum(v1 * uv, axis=1, keepdims=True) + 1e-6
        m = jnp.maximum(vu0, vu1)
        e0 = jnp.exp(vu0 - m)
        e1 = jnp.exp(vu1 - m)
        tot = e0 + e1
        a0 = e0 / tot
        a1 = e1 / tot
        comb_r[...] = a0 * xv[0] + a1 * xv[1]
        alpha_r[...] = jnp.concatenate([a0, a1], axis=1)

    return pl.pallas_call(
        body,
        grid=(G_BLK,),
        in_specs=[
            pl.BlockSpec((2, BN, H), lambda i: (0, i, 0)),
            pl.BlockSpec((H, H), lambda i: (0, 0)),
            pl.BlockSpec((1, H), lambda i: (0, 0)),
        ],
        out_specs=[
            pl.BlockSpec((BN, H), lambda i: (i, 0)),
            pl.BlockSpec((BN, 2), lambda i: (i, 0)),
        ],
        out_shape=[
            jax.ShapeDtypeStruct((N, H), jnp.float32),
            jax.ShapeDtypeStruct((N, 2), jnp.float32),
        ],
    )(x2, w_omega, u_row)


def _tc_stats1(comb, w1, bl1):
    def body(c_r, w_r, b_r, st_r, acc):
        i = pl.program_id(0)

        @pl.when(i == 0)
        def _():
            acc[...] = jnp.zeros_like(acc)

        y = _dot(c_r[...], w_r[...]) + b_r[...]
        acc[0, :] += jnp.sum(y, axis=0)
        acc[1, :] += jnp.sum(y * y, axis=0)

        @pl.when(i == G_BLK - 1)
        def _():
            st_r[...] = acc[...]

    return pl.pallas_call(
        body,
        grid=(G_BLK,),
        in_specs=[
            pl.BlockSpec((BN, H), lambda i: (i, 0)),
            pl.BlockSpec((H, 1024), lambda i: (0, 0)),
            pl.BlockSpec((1, 1024), lambda i: (0, 0)),
        ],
        out_specs=pl.BlockSpec((2, 1024), lambda i: (0, 0)),
        out_shape=jax.ShapeDtypeStruct((2, 1024), jnp.float32),
        scratch_shapes=[pltpu.VMEM((2, 1024), jnp.float32)],
    )(comb, w1, bl1)


def _tc_mlp1(comb, st1, w1, bl1, g1, beta1, w2, bl2):
    def body(c_r, st_r, w1_r, b1_r, g1_r, be1_r, w2_r, b2_r, y2_r, st2_r, acc):
        i = pl.program_id(0)

        @pl.when(i == 0)
        def _():
            acc[...] = jnp.zeros_like(acc)

        st = st_r[...]
        mean = st[0:1, :] / N
        var = st[1:2, :] / N - mean * mean
        scale = g1_r[...] * lax.rsqrt(var + 1e-5)
        shift = be1_r[...] - mean * scale
        y1 = _dot(c_r[...], w1_r[...]) + b1_r[...]
        x = jnp.maximum(y1 * scale + shift, 0.0)
        y2 = _dot(x, w2_r[...]) + b2_r[...]
        y2_r[...] = y2
        acc[0, :] += jnp.sum(y2, axis=0)
        acc[1, :] += jnp.sum(y2 * y2, axis=0)

        @pl.when(i == G_BLK - 1)
        def _():
            st2_r[...] = acc[...]

    return pl.pallas_call(
        body,
        grid=(G_BLK,),
        in_specs=[
            pl.BlockSpec((BN, H), lambda i: (i, 0)),
            pl.BlockSpec((2, 1024), lambda i: (0, 0)),
            pl.BlockSpec((H, 1024), lambda i: (0, 0)),
            pl.BlockSpec((1, 1024), lambda i: (0, 0)),
            pl.BlockSpec((1, 1024), lambda i: (0, 0)),
            pl.BlockSpec((1, 1024), lambda i: (0, 0)),
            pl.BlockSpec((1024, H), lambda i: (0, 0)),
            pl.BlockSpec((1, H), lambda i: (0, 0)),
        ],
        out_specs=[
            pl.BlockSpec((BN, H), lambda i: (i, 0)),
            pl.BlockSpec((2, H), lambda i: (0, 0)),
        ],
        out_shape=[
            jax.ShapeDtypeStruct((N, H), jnp.float32),
            jax.ShapeDtypeStruct((2, H), jnp.float32),
        ],
        scratch_shapes=[pltpu.VMEM((2, H), jnp.float32)],
    )(comb, st1, w1, bl1, g1, beta1, w2, bl2)


def _tc_mlp2(y2, st2, g2, beta2, w3, bl3):
    def body(y_r, st_r, g_r, be_r, w_r, b_r, out_r):
        st = st_r[...]
        mean = st[0:1, :] / N
        var = st[1:2, :] / N - mean * mean
        scale = g_r[...] * lax.rsqrt(var + 1e-5)
        shift = be_r[...] - mean * scale
        x = jnp.maximum(y_r[...] * scale + shift, 0.0)
        out_r[...] = _dot(x, w_r[...]) + b_r[...]

    return pl.pallas_call(
        body,
        grid=(G_BLK,),
        in_specs=[
            pl.BlockSpec((BN, H), lambda i: (i, 0)),
            pl.BlockSpec((2, H), lambda i: (0, 0)),
            pl.BlockSpec((1, H), lambda i: (0, 0)),
            pl.BlockSpec((1, H), lambda i: (0, 0)),
            pl.BlockSpec((H, OUT), lambda i: (0, 0)),
            pl.BlockSpec((1, OUT), lambda i: (0, 0)),
        ],
        out_specs=pl.BlockSpec((BN, OUT), lambda i: (i, 0)),
        out_shape=jax.ShapeDtypeStruct((N, OUT), jnp.float32),
    )(y2, st2, g2, beta2, w3, bl3)


def _prep_edges(edge):
    src = jnp.concatenate([edge[0], jnp.zeros((E_PAD - E,), jnp.int32)])
    dst = jnp.concatenate([edge[1], jnp.full((E_PAD - E,), N, jnp.int32)])
    return src.reshape(NS * K, CH), dst.reshape(NS * K, CH)


def kernel(feat_omics1, edge_index_spatial, edge_index_feature, weight,
           Wself0, Wneigh0, b0, Wself1, Wneigh1, b1c, w_omega, u_omega,
           W1, bl1, g1, beta1, W2, bl2, g2, beta2, W3, bl3):
    zb = jnp.zeros((N_PAD, D), jnp.float32)
    ones_h = jnp.ones((CH, D), jnp.float32)

    src_s0, dst_s = _prep_edges(edge_index_spatial)
    src_f0, dst_f = _prep_edges(edge_index_feature)
    src_l0 = jnp.stack([src_s0, src_f0])
    src_l1 = jnp.stack([src_s0, src_f0 + N])
    dst_all = jnp.stack([dst_s, dst_f])

    b0r = b0.reshape(1, H)
    b1r = b1c.reshape(1, H)
    u_row = u_omega.reshape(1, H)
    bl1r = bl1.reshape(1, 1024)
    g1r = g1.reshape(1, 1024)
    beta1r = beta1.reshape(1, 1024)
    bl2r = bl2.reshape(1, H)
    g2r = g2.reshape(1, H)
    beta2r = beta2.reshape(1, H)
    bl3r = bl3.reshape(1, OUT)

    feat2 = jnp.concatenate([feat_omics1, feat_omics1])
    deg = _sc_dual_deg(dst_all, zb, ones_h)
    acc0 = _sc_dual_segsum(feat2, src_l1, dst_all, zb)
    acc0 = acc0[:, :N, :]
    degN = deg[:, :N, :16]
    x1 = _tc_sage(feat_omics1, acc0, degN, Wself0, Wneigh0, b0r, False)

    table1 = x1.reshape(2 * N, H)
    acc1 = _sc_dual_segsum(table1, src_l1, dst_all, zb)
    acc1 = acc1[:, :N, :]
    comb, alpha, st1 = _tc_sage1_att(x1, acc1, degN, Wself1, Wneigh1, b1r,
                                     w_omega, u_row, W1, bl1r)
    y2, st2 = _tc_mlp1(comb, st1, W1, bl1r, g1r, beta1r, W2, bl2r)
    out = _tc_mlp2(y2, st2, g2r, beta2r, W3, bl3r)
    return (out, alpha)


# speedup vs baseline: 1.1568x; 1.0682x over previous
import functools

import jax
import jax.numpy as jnp
from jax import lax
from jax.experimental import pallas as pl
from jax.experimental.pallas import tpu as pltpu
from jax.experimental.pallas import tpu_sc as plsc

N = 10000
E = 320000
D = 128
H = 128
OUT = 64

NS = 16
CH = 128
K = 160
KI = 32
E_PAD = NS * K * CH
N_PAD = 10240
RPT = N_PAD // NS
PIECES = RPT // CH
BN = 1000
G_BLK = N // BN


def _sc_mesh():
    return plsc.VectorSubcoreMesh(core_axis_name="c", subcore_axis_name="s")


def _sc_dual_segsum(table, src_i, dst_i, zb):
    @functools.partial(
        pl.kernel,
        out_type=jax.ShapeDtypeStruct((2, N_PAD, D), jnp.float32),
        mesh=_sc_mesh(),
        scratch_types=[
            pltpu.VMEM((KI, CH), jnp.int32),
            pltpu.VMEM((KI, CH), jnp.int32),
            pltpu.VMEM((CH, D), jnp.float32),
            pltpu.VMEM((CH, D), jnp.float32),
            pltpu.SemaphoreType.DMA,
            pltpu.SemaphoreType.DMA,
            pltpu.VMEM_SHARED((N_PAD, D), jnp.float32),
        ],
    )
    def k(table_h, src_h, dst_h, zb_h, out_h,
          src_v, dst_v, rows_a, rows_b, sem_a, sem_b, acc_sh):
        c = lax.axis_index("c")
        s = lax.axis_index("s")
        rs = s * RPT

        @pl.loop(0, PIECES)
        def _(p):
            off = rs + p * CH
            pltpu.sync_copy(zb_h.at[pl.ds(off, CH)], acc_sh.at[pl.ds(off, CH)])

        plsc.subcore_barrier()

        @pl.loop(0, K // KI)
        def _(o):
            pltpu.sync_copy(src_h.at[c, pl.ds(s * K + o * KI, KI)], src_v)
            pltpu.sync_copy(dst_h.at[c, pl.ds(s * K + o * KI, KI)], dst_v)
            pltpu.async_copy(table_h.at[src_v.at[0]], rows_a, sem_a)

            @pl.loop(0, KI // 2)
            def _(jj):
                j0 = 2 * jj
                pltpu.make_async_copy(table_h.at[src_v.at[j0]], rows_a, sem_a).wait()
                pltpu.async_copy(table_h.at[src_v.at[j0 + 1]], rows_b, sem_b)
                pltpu.sync_copy(rows_a, acc_sh.at[dst_v.at[j0]], add=True)
                pltpu.make_async_copy(table_h.at[src_v.at[j0 + 1]], rows_b, sem_b).wait()

                @pl.when(jj < KI // 2 - 1)
                def _():
                    pltpu.async_copy(table_h.at[src_v.at[j0 + 2]], rows_a, sem_a)

                pltpu.sync_copy(rows_b, acc_sh.at[dst_v.at[j0 + 1]], add=True)

        plsc.subcore_barrier()

        @pl.loop(0, PIECES)
        def _(p):
            off = rs + p * CH
            pltpu.sync_copy(acc_sh.at[pl.ds(off, CH)], out_h.at[c, pl.ds(off, CH)])

    return k(table, src_i, dst_i, zb)


def _sc_dual_deg(dst_i, zb, ones_h):
    @functools.partial(
        pl.kernel,
        out_type=jax.ShapeDtypeStruct((2, N_PAD, D), jnp.float32),
        mesh=_sc_mesh(),
        scratch_types=[
            pltpu.VMEM((KI, CH), jnp.int32),
            pltpu.VMEM((CH, D), jnp.float32),
            pltpu.VMEM_SHARED((N_PAD, D), jnp.float32),
        ],
    )
    def k(dst_h, zb_h, ones_hh, out_h, dst_v, ones_v, deg_sh):
        c = lax.axis_index("c")
        s = lax.axis_index("s")
        rs = s * RPT

        @pl.loop(0, PIECES)
        def _(p):
            off = rs + p * CH
            pltpu.sync_copy(zb_h.at[pl.ds(off, CH)], deg_sh.at[pl.ds(off, CH)])

        pltpu.sync_copy(ones_hh, ones_v)
        plsc.subcore_barrier()

        @pl.loop(0, K // KI)
        def _(o):
            pltpu.sync_copy(dst_h.at[c, pl.ds(s * K + o * KI, KI)], dst_v)

            @pl.loop(0, KI)
            def _(j):
                pltpu.sync_copy(ones_v, deg_sh.at[dst_v.at[j]], add=True)

        plsc.subcore_barrier()

        @pl.loop(0, PIECES)
        def _(p):
            off = rs + p * CH
            pltpu.sync_copy(deg_sh.at[pl.ds(off, CH)], out_h.at[c, pl.ds(off, CH)])

    return k(dst_i, zb, ones_h)


def _dot(a, b):
    return jnp.dot(a, b, preferred_element_type=jnp.float32)


def _tc_sage(x, acc, deg, wself, wneigh, b, residual):
    def body(x_r, acc_r, deg_r, ws_r, wn_r, b_r, out_r):
        xv = x_r[...]
        av = acc_r[...]
        dv = deg_r[...]
        ws = ws_r[...]
        wn = wn_r[...]
        bv = b_r[...]
        outs = []
        if xv.ndim == 2:
            fs = _dot(xv, ws)
            for g in range(2):
                hn = av[g] / jnp.maximum(dv[g, :, 0:1], 1.0)
                outs.append(fs + _dot(hn, wn) + bv)
        else:
            for g in range(2):
                hn = av[g] / jnp.maximum(dv[g, :, 0:1], 1.0)
                o = _dot(xv[g], ws) + _dot(hn, wn) + bv
                if residual:
                    o = o + xv[g]
                outs.append(o)
        out_r[...] = jnp.stack(outs)

    x_spec = (pl.BlockSpec((BN, D), lambda i: (i, 0)) if x.ndim == 2
              else pl.BlockSpec((2, BN, D), lambda i: (0, i, 0)))
    return pl.pallas_call(
        body,
        grid=(G_BLK,),
        in_specs=[
            x_spec,
            pl.BlockSpec((2, BN, D), lambda i: (0, i, 0)),
            pl.BlockSpec((2, BN, 128), lambda i: (0, i, 0)),
            pl.BlockSpec((D, H), lambda i: (0, 0)),
            pl.BlockSpec((D, H), lambda i: (0, 0)),
            pl.BlockSpec((1, H), lambda i: (0, 0)),
        ],
        out_specs=pl.BlockSpec((2, BN, H), lambda i: (0, i, 0)),
        out_shape=jax.ShapeDtypeStruct((2, N, H), jnp.float32),
    )(x, acc, deg, wself, wneigh, b)


def _tc_sage1_att(x1, acc1, deg, wself, wneigh, b, w_omega, u_row, w1, bl1):
    def body(x_r, acc_r, deg_r, ws_r, wn_r, b_r, wo_r, u_r, w1_r, b1_r,
             comb_r, alpha_r, st_r, accsc):
        i = pl.program_id(0)

        @pl.when(i == 0)
        def _():
            accsc[...] = jnp.zeros_like(accsc)

        xv = x_r[...]
        av = acc_r[...]
        dv = deg_r[...]
        ws = ws_r[...]
        wn = wn_r[...]
        bv = b_r[...]
        x2 = []
        for g in range(2):
            hn = av[g] / jnp.maximum(dv[g, :, 0:1], 1.0)
            x2.append(_dot(xv[g], ws) + _dot(hn, wn) + bv + xv[g])
        wo = wo_r[...]
        uv = u_r[...]
        v0 = jnp.tanh(_dot(x2[0], wo))
        v1 = jnp.tanh(_dot(x2[1], wo))
        vu0 = jnp.sum(v0 * uv, axis=1, keepdims=True) + 1e-6
        vu1 = jnp.sum(v1 * uv, axis=1, keepdims=True) + 1e-6
        m = jnp.maximum(vu0, vu1)
        e0 = jnp.exp(vu0 - m)
        e1 = jnp.exp(vu1 - m)
        tot = e0 + e1
        a0 = e0 / tot
        a1 = e1 / tot
        comb = a0 * x2[0] + a1 * x2[1]
        comb_r[...] = comb
        alpha_r[...] = jnp.concatenate([a0, a1], axis=1)
        y = _dot(comb, w1_r[...]) + b1_r[...]
        accsc[0, :] += jnp.sum(y, axis=0)
        accsc[1, :] += jnp.sum(y * y, axis=0)

        @pl.when(i == G_BLK - 1)
        def _():
            st_r[...] = accsc[...]

    return pl.pallas_call(
        body,
        grid=(G_BLK,),
        in_specs=[
            pl.BlockSpec((2, BN, H), lambda i: (0, i, 0)),
            pl.BlockSpec((2, BN, H), lambda i: (0, i, 0)),
            pl.BlockSpec((2, BN, 128), lambda i: (0, i, 0)),
            pl.BlockSpec((H, H), lambda i: (0, 0)),
            pl.BlockSpec((H, H), lambda i: (0, 0)),
            pl.BlockSpec((1, H), lambda i: (0, 0)),
            pl.BlockSpec((H, H), lambda i: (0, 0)),
            pl.BlockSpec((1, H), lambda i: (0, 0)),
            pl.BlockSpec((H, 1024), lambda i: (0, 0)),
            pl.BlockSpec((1, 1024), lambda i: (0, 0)),
        ],
        out_specs=[
            pl.BlockSpec((BN, H), lambda i: (i, 0)),
            pl.BlockSpec((BN, 2), lambda i: (i, 0)),
            pl.BlockSpec((2, 1024), lambda i: (0, 0)),
        ],
        out_shape=[
            jax.ShapeDtypeStruct((N, H), jnp.float32),
            jax.ShapeDtypeStruct((N, 2), jnp.float32),
            jax.ShapeDtypeStruct((2, 1024), jnp.float32),
        ],
        scratch_shapes=[pltpu.VMEM((2, 1024), jnp.float32)],
    )(x1, acc1, deg, wself, wneigh, b, w_omega, u_row, w1, bl1)


def _tc_attention(x2, w_omega, u_row):
    def body(x_r, wo_r, u_r, comb_r, alpha_r):
        xv = x_r[...]
        wo = wo_r[...]
        uv = u_r[...]
        v0 = jnp.tanh(_dot(xv[0], wo))
        v1 = jnp.tanh(_dot(xv[1], wo))
        vu0 = jnp.sum(v0 * uv, axis=1, keepdims=True) + 1e-6
        vu1 = jnp.sum(v1 * uv, axis=1, keepdims=True) + 1e-6
        m = jnp.maximum(vu0, vu1)
        e0 = jnp.exp(vu0 - m)
        e1 = jnp.exp(vu1 - m)
        tot = e0 + e1
        a0 = e0 / tot
        a1 = e1 / tot
        comb_r[...] = a0 * xv[0] + a1 * xv[1]
        alpha_r[...] = jnp.concatenate([a0, a1], axis=1)

    return pl.pallas_call(
        body,
        grid=(G_BLK,),
        in_specs=[
            pl.BlockSpec((2, BN, H), lambda i: (0, i, 0)),
            pl.BlockSpec((H, H), lambda i: (0, 0)),
            pl.BlockSpec((1, H), lambda i: (0, 0)),
        ],
        out_specs=[
            pl.BlockSpec((BN, H), lambda i: (i, 0)),
            pl.BlockSpec((BN, 2), lambda i: (i, 0)),
        ],
        out_shape=[
            jax.ShapeDtypeStruct((N, H), jnp.float32),
            jax.ShapeDtypeStruct((N, 2), jnp.float32),
        ],
    )(x2, w_omega, u_row)


def _tc_stats1(comb, w1, bl1):
    def body(c_r, w_r, b_r, st_r, acc):
        i = pl.program_id(0)

        @pl.when(i == 0)
        def _():
            acc[...] = jnp.zeros_like(acc)

        y = _dot(c_r[...], w_r[...]) + b_r[...]
        acc[0, :] += jnp.sum(y, axis=0)
        acc[1, :] += jnp.sum(y * y, axis=0)

        @pl.when(i == G_BLK - 1)
        def _():
            st_r[...] = acc[...]

    return pl.pallas_call(
        body,
        grid=(G_BLK,),
        in_specs=[
            pl.BlockSpec((BN, H), lambda i: (i, 0)),
            pl.BlockSpec((H, 1024), lambda i: (0, 0)),
            pl.BlockSpec((1, 1024), lambda i: (0, 0)),
        ],
        out_specs=pl.BlockSpec((2, 1024), lambda i: (0, 0)),
        out_shape=jax.ShapeDtypeStruct((2, 1024), jnp.float32),
        scratch_shapes=[pltpu.VMEM((2, 1024), jnp.float32)],
    )(comb, w1, bl1)


def _tc_mlp1(comb, st1, w1, bl1, g1, beta1, w2, bl2):
    def body(c_r, st_r, w1_r, b1_r, g1_r, be1_r, w2_r, b2_r, y2_r, st2_r, acc):
        i = pl.program_id(0)

        @pl.when(i == 0)
        def _():
            acc[...] = jnp.zeros_like(acc)

        st = st_r[...]
        mean = st[0:1, :] / N
        var = st[1:2, :] / N - mean * mean
        scale = g1_r[...] * lax.rsqrt(var + 1e-5)
        shift = be1_r[...] - mean * scale
        y1 = _dot(c_r[...], w1_r[...]) + b1_r[...]
        x = jnp.maximum(y1 * scale + shift, 0.0)
        y2 = _dot(x, w2_r[...]) + b2_r[...]
        y2_r[...] = y2
        acc[0, :] += jnp.sum(y2, axis=0)
        acc[1, :] += jnp.sum(y2 * y2, axis=0)

        @pl.when(i == G_BLK - 1)
        def _():
            st2_r[...] = acc[...]

    return pl.pallas_call(
        body,
        grid=(G_BLK,),
        in_specs=[
            pl.BlockSpec((BN, H), lambda i: (i, 0)),
            pl.BlockSpec((2, 1024), lambda i: (0, 0)),
            pl.BlockSpec((H, 1024), lambda i: (0, 0)),
            pl.BlockSpec((1, 1024), lambda i: (0, 0)),
            pl.BlockSpec((1, 1024), lambda i: (0, 0)),
            pl.BlockSpec((1, 1024), lambda i: (0, 0)),
            pl.BlockSpec((1024, H), lambda i: (0, 0)),
            pl.BlockSpec((1, H), lambda i: (0, 0)),
        ],
        out_specs=[
            pl.BlockSpec((BN, H), lambda i: (i, 0)),
            pl.BlockSpec((2, H), lambda i: (0, 0)),
        ],
        out_shape=[
            jax.ShapeDtypeStruct((N, H), jnp.float32),
            jax.ShapeDtypeStruct((2, H), jnp.float32),
        ],
        scratch_shapes=[pltpu.VMEM((2, H), jnp.float32)],
    )(comb, st1, w1, bl1, g1, beta1, w2, bl2)


def _tc_mlp2(y2, st2, g2, beta2, w3, bl3):
    def body(y_r, st_r, g_r, be_r, w_r, b_r, out_r):
        st = st_r[...]
        mean = st[0:1, :] / N
        var = st[1:2, :] / N - mean * mean
        scale = g_r[...] * lax.rsqrt(var + 1e-5)
        shift = be_r[...] - mean * scale
        x = jnp.maximum(y_r[...] * scale + shift, 0.0)
        out_r[...] = _dot(x, w_r[...]) + b_r[...]

    return pl.pallas_call(
        body,
        grid=(G_BLK,),
        in_specs=[
            pl.BlockSpec((BN, H), lambda i: (i, 0)),
            pl.BlockSpec((2, H), lambda i: (0, 0)),
            pl.BlockSpec((1, H), lambda i: (0, 0)),
            pl.BlockSpec((1, H), lambda i: (0, 0)),
            pl.BlockSpec((H, OUT), lambda i: (0, 0)),
            pl.BlockSpec((1, OUT), lambda i: (0, 0)),
        ],
        out_specs=pl.BlockSpec((BN, OUT), lambda i: (i, 0)),
        out_shape=jax.ShapeDtypeStruct((N, OUT), jnp.float32),
    )(y2, st2, g2, beta2, w3, bl3)


def _prep_edges(edge):
    src = jnp.concatenate([edge[0], jnp.zeros((E_PAD - E,), jnp.int32)])
    dst = jnp.concatenate([edge[1], jnp.full((E_PAD - E,), N, jnp.int32)])
    return src.reshape(NS * K, CH), dst.reshape(NS * K, CH)


def kernel(feat_omics1, edge_index_spatial, edge_index_feature, weight,
           Wself0, Wneigh0, b0, Wself1, Wneigh1, b1c, w_omega, u_omega,
           W1, bl1, g1, beta1, W2, bl2, g2, beta2, W3, bl3):
    zb = jnp.zeros((N_PAD, D), jnp.float32)
    ones_h = jnp.ones((CH, D), jnp.float32)

    src_s0, dst_s = _prep_edges(edge_index_spatial)
    src_f0, dst_f = _prep_edges(edge_index_feature)
    src_l0 = jnp.stack([src_s0, src_f0])
    src_l1 = jnp.stack([src_s0, src_f0 + N])
    dst_all = jnp.stack([dst_s, dst_f])

    b0r = b0.reshape(1, H)
    b1r = b1c.reshape(1, H)
    u_row = u_omega.reshape(1, H)
    bl1r = bl1.reshape(1, 1024)
    g1r = g1.reshape(1, 1024)
    beta1r = beta1.reshape(1, 1024)
    bl2r = bl2.reshape(1, H)
    g2r = g2.reshape(1, H)
    beta2r = beta2.reshape(1, H)
    bl3r = bl3.reshape(1, OUT)

    feat2 = jnp.concatenate([feat_omics1, feat_omics1])
    degN = _sc_dual_deg(dst_all, zb, ones_h)
    acc0 = _sc_dual_segsum(feat2, src_l1, dst_all, zb)
    x1 = _tc_sage(feat_omics1, acc0, degN, Wself0, Wneigh0, b0r, False)

    table1 = x1.reshape(2 * N, H)
    acc1 = _sc_dual_segsum(table1, src_l1, dst_all, zb)
    comb, alpha, st1 = _tc_sage1_att(x1, acc1, degN, Wself1, Wneigh1, b1r,
                                     w_omega, u_row, W1, bl1r)
    y2, st2 = _tc_mlp1(comb, st1, W1, bl1r, g1r, beta1r, W2, bl2r)
    out = _tc_mlp2(y2, st2, g2r, beta2r, W3, bl3r)
    return (out, alpha)
